# baseline (speedup 1.0000x reference)
import functools

import jax
import jax.numpy as jnp
from jax import lax
from jax.experimental import pallas as pl
from jax.experimental.pallas import tpu as pltpu
from jax.experimental.pallas import tpu_sc as plsc

_LANES = 16


def _bcast(ref, idx):
    return plsc.load_gather(ref, [jnp.full((_LANES,), idx, jnp.int32)])


def _sc_body(num_cores, rows_per_worker, x_hbm, w_hbm, b_hbm, r_hbm, y_hbm,
             x_v, w_v, b_v, r_v, y_v, sem0, sem1, sem_o):
    wid = lax.axis_index("s") * num_cores + lax.axis_index("c")
    base = wid * rows_per_worker
    half = rows_per_worker // 2

    cps0 = [
        pltpu.async_copy(w_hbm, w_v, sem0),
        pltpu.async_copy(b_hbm, b_v, sem0),
        pltpu.async_copy(r_hbm, r_v, sem0),
        pltpu.async_copy(
            x_hbm.at[pl.ds(base * 4, half * 4)],
            x_v.at[pl.ds(0, half * 4)], sem0,
        ),
    ]
    cp1 = pltpu.async_copy(
        x_hbm.at[pl.ds((base + half) * 4, half * 4)],
        x_v.at[pl.ds(half * 4, half * 4)], sem1,
    )
    for cp in cps0:
        cp.wait()

    rb = [-_bcast(r_v, o) for o in range(2)]
    coef = [[rb[o] * _bcast(w_v, 28 + 4 * o + c) for c in range(4)]
            for o in range(2)]
    bias = [-_bcast(b_v, o) for o in range(2)]

    iota = lax.iota(jnp.int32, _LANES)

    def compute(lo_step, hi_step):
        @plsc.parallel_loop(lo_step, hi_step, unroll=2)
        def step(i):
            row = i * _LANES + iota
            cols = [plsc.load_gather(x_v, [row * 4 + c]) for c in range(4)]
            for o in range(2):
                acc = bias[o]
                for c in range(4):
                    acc = acc + coef[o][c] * cols[c]
                out = 1.0 / (1.0 + jnp.exp(acc))
                plsc.store_scatter(y_v, [row * 2 + o], out)

    half_steps = half // _LANES
    compute(0, half_steps)
    out0 = pltpu.async_copy(
        y_v.at[pl.ds(0, half * 2)],
        y_hbm.at[pl.ds(base * 2, half * 2)], sem_o,
    )
    cp1.wait()
    compute(half_steps, 2 * half_steps)
    out1 = pltpu.async_copy(
        y_v.at[pl.ds(half * 2, half * 2)],
        y_hbm.at[pl.ds((base + half) * 2, half * 2)], sem_o,
    )
    out0.wait()
    out1.wait()


def kernel(inputs, W, b, r):
    batch = inputs.shape[0]
    info = plsc.get_sparse_core_info()
    num_workers = info.num_cores * info.num_subcores
    rows_per_worker = batch // num_workers

    mesh = plsc.VectorSubcoreMesh(core_axis_name="c", subcore_axis_name="s")
    run = pl.kernel(
        functools.partial(_sc_body, info.num_cores, rows_per_worker),
        out_type=jax.ShapeDtypeStruct((batch * 2,), jnp.float32),
        mesh=mesh,
        compiler_params=pltpu.CompilerParams(needs_layout_passes=False),
        scratch_types=[
            pltpu.VMEM((rows_per_worker * 4,), jnp.float32),
            pltpu.VMEM((W.shape[0],), jnp.float32),
            pltpu.VMEM((b.shape[0],), jnp.float32),
            pltpu.VMEM((r.shape[0],), jnp.float32),
            pltpu.VMEM((rows_per_worker * 2,), jnp.float32),
            pltpu.SemaphoreType.DMA,
            pltpu.SemaphoreType.DMA,
            pltpu.SemaphoreType.DMA,
        ],
    )
    return run(inputs.reshape(-1), W, b, r).reshape(batch, 2)


# speedup vs baseline: 1.0026x; 1.0026x over previous
import functools

import jax
import jax.numpy as jnp
from jax import lax
from jax.experimental import pallas as pl
from jax.experimental.pallas import tpu as pltpu
from jax.experimental.pallas import tpu_sc as plsc

_LANES = 16


def _bcast(ref, idx):
    return plsc.load_gather(ref, [jnp.full((_LANES,), idx, jnp.int32)])


def _sc_body(num_cores, rows_per_worker, x_hbm, w_hbm, b_hbm, r_hbm, y_hbm,
             x_v, w_v, b_v, r_v, y_v, sem):
    wid = lax.axis_index("s") * num_cores + lax.axis_index("c")
    base = wid * rows_per_worker

    cps = [
        pltpu.async_copy(w_hbm, w_v, sem),
        pltpu.async_copy(b_hbm, b_v, sem),
        pltpu.async_copy(r_hbm, r_v, sem),
        pltpu.async_copy(
            x_hbm.at[pl.ds(base * 4, rows_per_worker * 4)], x_v, sem
        ),
    ]
    for cp in cps:
        cp.wait()

    rb = [-_bcast(r_v, o) for o in range(2)]
    coef = [[rb[o] * _bcast(w_v, 28 + 4 * o + c) for c in range(4)]
            for o in range(2)]
    bias = [-_bcast(b_v, o) for o in range(2)]

    iota = lax.iota(jnp.int32, _LANES)
    init = ([iota * 4 + c for c in range(4)], [iota * 2 + o for o in range(2)])

    @plsc.parallel_loop(0, rows_per_worker // _LANES, unroll=2, carry=init)
    def step(i, idx):
        xidx, yidx = idx
        cols = [plsc.load_gather(x_v, [xidx[c]]) for c in range(4)]
        for o in range(2):
            acc = bias[o]
            for c in range(4):
                acc = acc + coef[o][c] * cols[c]
            out = 1.0 / (1.0 + jnp.exp(acc))
            plsc.store_scatter(y_v, [yidx[o]], out)
        return ([v + (4 * _LANES) for v in xidx],
                [v + (2 * _LANES) for v in yidx])

    pltpu.sync_copy(y_v, y_hbm.at[pl.ds(base * 2, rows_per_worker * 2)])


def kernel(inputs, W, b, r):
    batch = inputs.shape[0]
    info = plsc.get_sparse_core_info()
    num_workers = info.num_cores * info.num_subcores
    rows_per_worker = batch // num_workers

    mesh = plsc.VectorSubcoreMesh(core_axis_name="c", subcore_axis_name="s")
    run = pl.kernel(
        functools.partial(_sc_body, info.num_cores, rows_per_worker),
        out_type=jax.ShapeDtypeStruct((batch * 2,), jnp.float32),
        mesh=mesh,
        compiler_params=pltpu.CompilerParams(needs_layout_passes=False),
        scratch_types=[
            pltpu.VMEM((rows_per_worker * 4,), jnp.float32),
            pltpu.VMEM((W.shape[0],), jnp.float32),
            pltpu.VMEM((b.shape[0],), jnp.float32),
            pltpu.VMEM((r.shape[0],), jnp.float32),
            pltpu.VMEM((rows_per_worker * 2,), jnp.float32),
            pltpu.SemaphoreType.DMA,
        ],
    )
    return run(inputs.reshape(-1), W, b, r).reshape(batch, 2)


# speedup vs baseline: 1.0032x; 1.0006x over previous
import functools

import jax
import jax.numpy as jnp
from jax import lax
from jax.experimental import pallas as pl
from jax.experimental.pallas import tpu as pltpu
from jax.experimental.pallas import tpu_sc as plsc

_LANES = 16


def _bcast(ref, idx):
    return plsc.load_gather(ref, [jnp.full((_LANES,), idx, jnp.int32)])


def _sc_body(num_cores, rows_per_worker, x_hbm, w_hbm, b_hbm, r_hbm, y_hbm,
             x_v, w_v, b_v, r_v, y_v, sem, sem_x):
    wid = lax.axis_index("s") * num_cores + lax.axis_index("c")
    base = wid * rows_per_worker

    cp_x = pltpu.async_copy(
        x_hbm.at[pl.ds(base * 4, rows_per_worker * 4)], x_v, sem_x
    )
    cps = [
        pltpu.async_copy(w_hbm, w_v, sem),
        pltpu.async_copy(b_hbm, b_v, sem),
        pltpu.async_copy(r_hbm, r_v, sem),
    ]
    for cp in cps:
        cp.wait()

    rb = [-_bcast(r_v, o) for o in range(2)]
    coef = [[rb[o] * _bcast(w_v, 28 + 4 * o + c) for c in range(4)]
            for o in range(2)]
    bias = [-_bcast(b_v, o) for o in range(2)]

    cp_x.wait()

    iota = lax.iota(jnp.int32, _LANES)
    init = ([iota * 4 + c for c in range(4)], [iota * 2 + o for o in range(2)])

    @plsc.parallel_loop(0, rows_per_worker // _LANES, unroll=2, carry=init)
    def step(i, idx):
        xidx, yidx = idx
        cols = [plsc.load_gather(x_v, [xidx[c]]) for c in range(4)]
        for o in range(2):
            acc = bias[o]
            for c in range(4):
                acc = acc + coef[o][c] * cols[c]
            out = 1.0 / (1.0 + jnp.exp(acc))
            plsc.store_scatter(y_v, [yidx[o]], out)
        return ([v + (4 * _LANES) for v in xidx],
                [v + (2 * _LANES) for v in yidx])

    pltpu.sync_copy(y_v, y_hbm.at[pl.ds(base * 2, rows_per_worker * 2)])


def kernel(inputs, W, b, r):
    batch = inputs.shape[0]
    info = plsc.get_sparse_core_info()
    num_workers = info.num_cores * info.num_subcores
    rows_per_worker = batch // num_workers

    mesh = plsc.VectorSubcoreMesh(core_axis_name="c", subcore_axis_name="s")
    run = pl.kernel(
        functools.partial(_sc_body, info.num_cores, rows_per_worker),
        out_type=jax.ShapeDtypeStruct((batch * 2,), jnp.float32),
        mesh=mesh,
        compiler_params=pltpu.CompilerParams(needs_layout_passes=False),
        scratch_types=[
            pltpu.VMEM((rows_per_worker * 4,), jnp.float32),
            pltpu.VMEM((W.shape[0],), jnp.float32),
            pltpu.VMEM((b.shape[0],), jnp.float32),
            pltpu.VMEM((r.shape[0],), jnp.float32),
            pltpu.VMEM((rows_per_worker * 2,), jnp.float32),
            pltpu.SemaphoreType.DMA,
            pltpu.SemaphoreType.DMA,
        ],
    )
    return run(inputs.reshape(-1), W, b, r).reshape(batch, 2)


# speedup vs baseline: 1.0102x; 1.0069x over previous
import functools

import jax
import jax.numpy as jnp
from jax import lax
from jax.experimental import pallas as pl
from jax.experimental.pallas import tpu as pltpu
from jax.experimental.pallas import tpu_sc as plsc

_LANES = 16


def _bcast(ref, idx):
    return plsc.load_gather(ref, [jnp.full((_LANES,), idx, jnp.int32)])


def _sc_body(num_cores, rows_per_worker, x_hbm, p_hbm, y_hbm,
             x_v, p_v, y_v, sem, sem_x):
    wid = lax.axis_index("s") * num_cores + lax.axis_index("c")
    base = wid * rows_per_worker

    cp_x = pltpu.async_copy(
        x_hbm.at[pl.ds(base * 4, rows_per_worker * 4)], x_v, sem_x
    )
    pltpu.async_copy(p_hbm.at[pl.ds(0, 40)], p_v, sem).wait()

    rb = [-_bcast(p_v, 38 + o) for o in range(2)]
    coef = [[rb[o] * _bcast(p_v, 28 + 4 * o + c) for c in range(4)]
            for o in range(2)]
    bias = [-_bcast(p_v, 36 + o) for o in range(2)]

    cp_x.wait()

    iota = lax.iota(jnp.int32, _LANES)
    init = ([iota * 4 + c for c in range(4)], [iota * 2 + o for o in range(2)])

    @plsc.parallel_loop(0, rows_per_worker // _LANES, unroll=2, carry=init)
    def step(i, idx):
        xidx, yidx = idx
        cols = [plsc.load_gather(x_v, [xidx[c]]) for c in range(4)]
        for o in range(2):
            acc = bias[o]
            for c in range(4):
                acc = acc + coef[o][c] * cols[c]
            out = 1.0 / (1.0 + jnp.exp(acc))
            plsc.store_scatter(y_v, [yidx[o]], out)
        return ([v + (4 * _LANES) for v in xidx],
                [v + (2 * _LANES) for v in yidx])

    pltpu.sync_copy(y_v, y_hbm.at[pl.ds(base * 2, rows_per_worker * 2)])


def kernel(inputs, W, b, r):
    batch = inputs.shape[0]
    info = plsc.get_sparse_core_info()
    num_workers = info.num_cores * info.num_subcores
    rows_per_worker = batch // num_workers

    mesh = plsc.VectorSubcoreMesh(core_axis_name="c", subcore_axis_name="s")
    run = pl.kernel(
        functools.partial(_sc_body, info.num_cores, rows_per_worker),
        out_type=jax.ShapeDtypeStruct((batch * 2,), jnp.float32),
        mesh=mesh,
        compiler_params=pltpu.CompilerParams(needs_layout_passes=False),
        scratch_types=[
            pltpu.VMEM((rows_per_worker * 4,), jnp.float32),
            pltpu.VMEM((40,), jnp.float32),
            pltpu.VMEM((rows_per_worker * 2,), jnp.float32),
            pltpu.SemaphoreType.DMA,
            pltpu.SemaphoreType.DMA,
        ],
    )
    params = jnp.concatenate([W, b, r])
    return run(inputs.reshape(-1), params).reshape(batch, 2)


# speedup vs baseline: 1.0152x; 1.0050x over previous
import functools

import jax
import jax.numpy as jnp
from jax import lax
from jax.experimental import pallas as pl
from jax.experimental.pallas import tpu as pltpu
from jax.experimental.pallas import tpu_sc as plsc

_LANES = 16


def _bcast(ref, idx):
    return plsc.load_gather(ref, [jnp.full((_LANES,), idx, jnp.int32)])


def _sc_body(num_cores, rows_per_worker, x_hbm, p_hbm, y_hbm,
             x_v, p_v, y_v, sem, sem_x):
    wid = lax.axis_index("s") * num_cores + lax.axis_index("c")
    base = wid * rows_per_worker

    cp_x = pltpu.async_copy(
        x_hbm.at[pl.ds(base * 4, rows_per_worker * 4)], x_v, sem_x
    )
    pltpu.async_copy(p_hbm.at[pl.ds(0, 16)], p_v, sem).wait()

    rb = [-_bcast(p_v, 2 + o) for o in range(2)]
    coef = [[rb[o] * _bcast(p_v, 4 + 4 * o + c) for c in range(4)]
            for o in range(2)]
    bias = [-_bcast(p_v, o) for o in range(2)]

    cp_x.wait()

    iota = lax.iota(jnp.int32, _LANES)
    init = ([iota * 4 + c for c in range(4)], [iota * 2 + o for o in range(2)])

    @plsc.parallel_loop(0, rows_per_worker // _LANES, unroll=2, carry=init)
    def step(i, idx):
        xidx, yidx = idx
        cols = [plsc.load_gather(x_v, [xidx[c]]) for c in range(4)]
        for o in range(2):
            acc = bias[o]
            for c in range(4):
                acc = acc + coef[o][c] * cols[c]
            out = 1.0 / (1.0 + jnp.exp(acc))
            plsc.store_scatter(y_v, [yidx[o]], out)
        return ([v + (4 * _LANES) for v in xidx],
                [v + (2 * _LANES) for v in yidx])

    pltpu.sync_copy(y_v, y_hbm.at[pl.ds(base * 2, rows_per_worker * 2)])


def kernel(inputs, W, b, r):
    batch = inputs.shape[0]
    info = plsc.get_sparse_core_info()
    num_workers = info.num_cores * info.num_subcores
    rows_per_worker = batch // num_workers

    mesh = plsc.VectorSubcoreMesh(core_axis_name="c", subcore_axis_name="s")
    run = pl.kernel(
        functools.partial(_sc_body, info.num_cores, rows_per_worker),
        out_type=jax.ShapeDtypeStruct((batch * 2,), jnp.float32),
        mesh=mesh,
        compiler_params=pltpu.CompilerParams(needs_layout_passes=False),
        scratch_types=[
            pltpu.VMEM((rows_per_worker * 4,), jnp.float32),
            pltpu.VMEM((16,), jnp.float32),
            pltpu.VMEM((rows_per_worker * 2,), jnp.float32),
            pltpu.SemaphoreType.DMA,
            pltpu.SemaphoreType.DMA,
        ],
    )
    params = jnp.concatenate([b, r, W[28:36], jnp.zeros((4,), jnp.float32)])
    return run(inputs.reshape(-1), params).reshape(batch, 2)
